# baseline (speedup 1.0000x reference)
import functools

import jax
import jax.numpy as jnp
from jax import lax
from jax.experimental import pallas as pl
from jax.experimental.pallas import tpu as pltpu
from jax.experimental.pallas import tpu_sc as plsc

N = 2048
D = 768
H = 3072
E = 8
K = 2
B = 128
P = N * K + E * B
NB = P // B
NW = 32
L = 16


def _gelu_tanh(x):
    return 0.5 * x * (1.0 + jnp.tanh(jnp.sqrt(2.0 / jnp.pi) * (x + 0.044715 * x ** 3)))


def _gate_body(x_ref, wg_ref, bg_ref,
               pos0_ref, pos1_ref, w0_ref, w1_ref, be_ref):
    x = x_ref[...]
    logits = lax.dot_general(
        x, wg_ref[...], (((1,), (0,)), ((), ())),
        preferred_element_type=jnp.float32) + bg_ref[...]
    m = jnp.max(logits, axis=-1, keepdims=True)
    ex = jnp.exp(logits - m)
    p = ex / jnp.sum(ex, axis=-1, keepdims=True)

    iota8 = lax.broadcasted_iota(jnp.int32, (N, E), 1)
    m1 = jnp.max(p, axis=-1, keepdims=True)
    a1 = jnp.min(jnp.where(p == m1, iota8, E), axis=-1, keepdims=True)
    oh1 = iota8 == a1
    pm = jnp.where(oh1, -1.0, p)
    m2 = jnp.max(pm, axis=-1, keepdims=True)
    a2 = jnp.min(jnp.where(pm == m2, iota8, E), axis=-1, keepdims=True)
    oh2 = iota8 == a2

    w0_ref[...] = jnp.sum(jnp.where(oh1, p, 0.0), axis=-1, keepdims=True)
    w1_ref[...] = jnp.sum(jnp.where(oh2, p, 0.0), axis=-1, keepdims=True)

    oh = oh1.astype(jnp.int32) + oh2.astype(jnp.int32)
    c = oh
    k = 1
    while k < N:
        c = c + jnp.concatenate(
            [jnp.zeros((k, E), jnp.int32), c[:N - k, :]], axis=0)
        k *= 2
    cexc = c - oh
    counts = c[N - 1:N, :]

    pc = ((counts + (B - 1)) // B) * B
    ends = pc
    k = 1
    while k < E:
        ends = ends + jnp.concatenate(
            [jnp.zeros((1, k), jnp.int32), ends[:, :E - k]], axis=1)
        k *= 2
    opad = ends - pc

    r0 = jnp.sum(jnp.where(oh1, cexc, 0), axis=-1, keepdims=True)
    r1 = jnp.sum(jnp.where(oh2, cexc, 0), axis=-1, keepdims=True)
    off0 = jnp.sum(jnp.where(oh1, opad, 0), axis=-1, keepdims=True)
    off1 = jnp.sum(jnp.where(oh2, opad, 0), axis=-1, keepdims=True)
    pos0_ref[...] = off0 + r0
    pos1_ref[...] = off1 + r1

    bio = lax.broadcasted_iota(jnp.int32, (1, NB), 1) * B
    acc = jnp.zeros((1, NB), jnp.int32)
    for e in range(E):
        acc = acc + (bio >= ends[:, e:e + 1]).astype(jnp.int32)
    be_ref[...] = jnp.minimum(acc, E - 1)


def _scatter_body(p0_hbm, p1_hbm, w0_hbm, w1_hbm,
                  st_hbm, sw_hbm,
                  st_v, sw_v, p0_v, p1_v, w0_v, w1_v):
    wid = lax.axis_index("s") * 2 + lax.axis_index("c")

    @pl.when(wid == 0)
    def _():
        pltpu.sync_copy(p0_hbm, p0_v)
        pltpu.sync_copy(p1_hbm, p1_v)
        pltpu.sync_copy(w0_hbm, w0_v)
        pltpu.sync_copy(w1_hbm, w1_v)

        zi = jnp.zeros((L,), jnp.int32)
        zf = jnp.zeros((L,), jnp.float32)

        def zbody(i, carry):
            st_v[pl.ds(i * L, L)] = zi
            sw_v[pl.ds(i * L, L)] = zf
            return carry
        lax.fori_loop(0, P // L, zbody, 0)

        iota16 = lax.iota(jnp.int32, L)

        def sbody(i, carry):
            sl = pl.ds(i * L, L)
            tok = i * L + iota16
            idx0 = p0_v[sl]
            plsc.store_scatter(st_v, [idx0], tok)
            plsc.store_scatter(sw_v, [idx0], w0_v[sl])
            idx1 = p1_v[sl]
            plsc.store_scatter(st_v, [idx1], tok)
            plsc.store_scatter(sw_v, [idx1], w1_v[sl])
            return carry
        lax.fori_loop(0, N // L, sbody, 0)

        pltpu.sync_copy(st_v, st_hbm)
        pltpu.sync_copy(sw_v, sw_hbm)


_B_PER_W = P // NW


def _gather_body(x_hbm, st_hbm, xs_hbm, idx_v, rows_v, sem):
    wid = lax.axis_index("s") * 2 + lax.axis_index("c")
    base = wid * _B_PER_W
    pltpu.sync_copy(st_hbm.at[pl.ds(base, _B_PER_W)], idx_v)
    pltpu.async_copy(x_hbm.at[idx_v], rows_v, sem).wait()
    pltpu.sync_copy(rows_v, xs_hbm.at[pl.ds(base, _B_PER_W)])


def _moe_body(be_ref, xs_ref, w1_ref, b1_ref, w2_ref, b2_ref, sw_ref, ys_ref):
    xb = xs_ref[...].astype(jnp.bfloat16)
    h = lax.dot_general(
        xb, w1_ref[0], (((1,), (0,)), ((), ())),
        preferred_element_type=jnp.float32) + b1_ref[0]
    h = _gelu_tanh(h)
    y = lax.dot_general(
        h.astype(jnp.bfloat16), w2_ref[0], (((1,), (0,)), ((), ())),
        preferred_element_type=jnp.float32) + b2_ref[0]
    ys_ref[...] = y * sw_ref[...]


_T_PER_W = N // NW


def _combine_body(ys_hbm, p0_hbm, p1_hbm, out_hbm,
                  i0_v, i1_v, r0_v, r1_v, s0, s1):
    wid = lax.axis_index("s") * 2 + lax.axis_index("c")
    base = wid * _T_PER_W
    pltpu.sync_copy(p0_hbm.at[pl.ds(base, _T_PER_W)], i0_v)
    pltpu.sync_copy(p1_hbm.at[pl.ds(base, _T_PER_W)], i1_v)
    cp0 = pltpu.async_copy(ys_hbm.at[i0_v], r0_v, s0)
    cp1 = pltpu.async_copy(ys_hbm.at[i1_v], r1_v, s1)
    cp0.wait()
    cp1.wait()

    def tbody(t, carry):
        for j in range(D // L):
            sl = pl.ds(j * L, L)
            r0_v[t, sl] = r0_v[t, sl] + r1_v[t, sl]
        return carry
    lax.fori_loop(0, _T_PER_W, tbody, 0)

    pltpu.sync_copy(r0_v, out_hbm.at[pl.ds(base, _T_PER_W)])


def kernel(x, Wg, bg, W1, b1, W2, b2):
    f32 = jnp.float32
    i32 = jnp.int32

    pos0, pos1, w0, w1, be = pl.pallas_call(
        _gate_body,
        out_shape=[
            jax.ShapeDtypeStruct((N, 1), i32),
            jax.ShapeDtypeStruct((N, 1), i32),
            jax.ShapeDtypeStruct((N, 1), f32),
            jax.ShapeDtypeStruct((N, 1), f32),
            jax.ShapeDtypeStruct((1, NB), i32),
        ],
    )(x, Wg, bg.reshape(1, E))
    pos0 = pos0.reshape(N)
    pos1 = pos1.reshape(N)
    w0f = w0.reshape(N)
    w1f = w1.reshape(N)
    be = be.reshape(NB)

    mesh = plsc.VectorSubcoreMesh(core_axis_name="c", subcore_axis_name="s")
    sc_params = pltpu.CompilerParams(needs_layout_passes=False)

    scatter_k = pl.kernel(
        _scatter_body,
        out_type=(
            jax.ShapeDtypeStruct((P,), i32),
            jax.ShapeDtypeStruct((P,), f32),
        ),
        mesh=mesh,
        scratch_types=[
            pltpu.VMEM((P,), i32),
            pltpu.VMEM((P,), f32),
            pltpu.VMEM((N,), i32),
            pltpu.VMEM((N,), i32),
            pltpu.VMEM((N,), f32),
            pltpu.VMEM((N,), f32),
        ],
        compiler_params=sc_params,
    )
    sorted_tok, sorted_w = scatter_k(pos0, pos1, w0f, w1f)

    gather_k = pl.kernel(
        _gather_body,
        out_type=jax.ShapeDtypeStruct((P, D), f32),
        mesh=mesh,
        scratch_types=[
            pltpu.VMEM((_B_PER_W,), i32),
            pltpu.VMEM((_B_PER_W, D), f32),
            pltpu.SemaphoreType.DMA,
        ],
        compiler_params=sc_params,
    )
    xs = gather_k(x, sorted_tok)

    grid_spec = pltpu.PrefetchScalarGridSpec(
        num_scalar_prefetch=1,
        grid=(NB,),
        in_specs=[
            pl.BlockSpec((B, D), lambda i, be_s: (i, 0)),
            pl.BlockSpec((1, D, H), lambda i, be_s: (be_s[i], 0, 0)),
            pl.BlockSpec((1, 1, H), lambda i, be_s: (be_s[i], 0, 0)),
            pl.BlockSpec((1, H, D), lambda i, be_s: (be_s[i], 0, 0)),
            pl.BlockSpec((1, 1, D), lambda i, be_s: (be_s[i], 0, 0)),
            pl.BlockSpec((B, 1), lambda i, be_s: (i, 0)),
        ],
        out_specs=pl.BlockSpec((B, D), lambda i, be_s: (i, 0)),
    )
    ys = pl.pallas_call(
        _moe_body,
        grid_spec=grid_spec,
        out_shape=jax.ShapeDtypeStruct((P, D), f32),
    )(be, xs,
      W1.astype(jnp.bfloat16), b1.reshape(E, 1, H),
      W2.astype(jnp.bfloat16), b2.reshape(E, 1, D),
      sorted_w.reshape(P, 1))

    combine_k = pl.kernel(
        _combine_body,
        out_type=jax.ShapeDtypeStruct((N, D), f32),
        mesh=mesh,
        scratch_types=[
            pltpu.VMEM((_T_PER_W,), i32),
            pltpu.VMEM((_T_PER_W,), i32),
            pltpu.VMEM((_T_PER_W, D), f32),
            pltpu.VMEM((_T_PER_W, D), f32),
            pltpu.SemaphoreType.DMA,
            pltpu.SemaphoreType.DMA,
        ],
        compiler_params=sc_params,
    )
    return combine_k(ys, pos0, pos1)


# speedup vs baseline: 1.1592x; 1.1592x over previous
import functools

import jax
import jax.numpy as jnp
from jax import lax
from jax.experimental import pallas as pl
from jax.experimental.pallas import tpu as pltpu
from jax.experimental.pallas import tpu_sc as plsc

N = 2048
D = 768
H = 3072
E = 8
K = 2
B = 128
P = N * K + E * B
NB = P // B
NW = 32
L = 16


def _gelu_tanh(x):
    return 0.5 * x * (1.0 + jnp.tanh(jnp.sqrt(2.0 / jnp.pi) * (x + 0.044715 * x ** 3)))


def _gate_body(x_ref, wg_ref, bg_ref,
               pos0_ref, pos1_ref, w0_ref, w1_ref, be_ref):
    x = x_ref[...]
    logits = lax.dot_general(
        x, wg_ref[...], (((1,), (0,)), ((), ())),
        preferred_element_type=jnp.float32) + bg_ref[...]
    m = jnp.max(logits, axis=-1, keepdims=True)
    ex = jnp.exp(logits - m)
    p = ex / jnp.sum(ex, axis=-1, keepdims=True)

    iota8 = lax.broadcasted_iota(jnp.int32, (N, E), 1)
    m1 = jnp.max(p, axis=-1, keepdims=True)
    a1 = jnp.min(jnp.where(p == m1, iota8, E), axis=-1, keepdims=True)
    oh1 = iota8 == a1
    pm = jnp.where(oh1, -1.0, p)
    m2 = jnp.max(pm, axis=-1, keepdims=True)
    a2 = jnp.min(jnp.where(pm == m2, iota8, E), axis=-1, keepdims=True)
    oh2 = iota8 == a2

    w0_ref[...] = jnp.sum(jnp.where(oh1, p, 0.0), axis=-1, keepdims=True)
    w1_ref[...] = jnp.sum(jnp.where(oh2, p, 0.0), axis=-1, keepdims=True)

    oh = oh1.astype(jnp.int32) + oh2.astype(jnp.int32)
    c = oh
    k = 1
    while k < N:
        c = c + jnp.concatenate(
            [jnp.zeros((k, E), jnp.int32), c[:N - k, :]], axis=0)
        k *= 2
    cexc = c - oh
    counts = c[N - 1:N, :]

    pc = ((counts + (B - 1)) // B) * B
    ends = pc
    k = 1
    while k < E:
        ends = ends + jnp.concatenate(
            [jnp.zeros((1, k), jnp.int32), ends[:, :E - k]], axis=1)
        k *= 2
    opad = ends - pc

    r0 = jnp.sum(jnp.where(oh1, cexc, 0), axis=-1, keepdims=True)
    r1 = jnp.sum(jnp.where(oh2, cexc, 0), axis=-1, keepdims=True)
    off0 = jnp.sum(jnp.where(oh1, opad, 0), axis=-1, keepdims=True)
    off1 = jnp.sum(jnp.where(oh2, opad, 0), axis=-1, keepdims=True)
    pos0_ref[...] = off0 + r0
    pos1_ref[...] = off1 + r1

    bio = lax.broadcasted_iota(jnp.int32, (1, NB), 1) * B
    acc = jnp.zeros((1, NB), jnp.int32)
    for e in range(E):
        acc = acc + (bio >= ends[:, e:e + 1]).astype(jnp.int32)
    be_ref[...] = jnp.minimum(acc, E - 1)


def _moe_body(be_ref, p0_ref, p1_ref, w0_ref, w1_ref, x_ref,
              w1e_ref, b1_ref, w2e_ref, b2_ref, ys_ref):
    i = pl.program_id(0)
    sid = i * B + lax.broadcasted_iota(jnp.int32, (B, 1), 0)
    m0 = p0_ref[...] == sid
    m1 = p1_ref[...] == sid
    gm = (m0 | m1).astype(jnp.bfloat16)
    xb = lax.dot_general(
        gm, x_ref[...], (((1,), (0,)), ((), ())),
        preferred_element_type=jnp.float32).astype(jnp.bfloat16)
    sw = jnp.sum(jnp.where(m0, w0_ref[...], 0.0) +
                 jnp.where(m1, w1_ref[...], 0.0), axis=1, keepdims=True)
    h = lax.dot_general(
        xb, w1e_ref[0], (((1,), (0,)), ((), ())),
        preferred_element_type=jnp.float32) + b1_ref[0]
    h = _gelu_tanh(h)
    y = lax.dot_general(
        h.astype(jnp.bfloat16), w2e_ref[0], (((1,), (0,)), ((), ())),
        preferred_element_type=jnp.float32) + b2_ref[0]
    ys_ref[...] = y * sw


_T_PER_W = N // NW


def _combine_body(ys_hbm, p0_hbm, p1_hbm, out_hbm,
                  i0_v, i1_v, r0_v, r1_v, s0, s1):
    wid = lax.axis_index("s") * 2 + lax.axis_index("c")
    base = wid * _T_PER_W
    pltpu.sync_copy(p0_hbm.at[pl.ds(base, _T_PER_W)], i0_v)
    pltpu.sync_copy(p1_hbm.at[pl.ds(base, _T_PER_W)], i1_v)
    cp0 = pltpu.async_copy(ys_hbm.at[i0_v], r0_v, s0)
    cp1 = pltpu.async_copy(ys_hbm.at[i1_v], r1_v, s1)
    cp0.wait()
    cp1.wait()

    def tbody(t, carry):
        for j in range(D // L):
            sl = pl.ds(j * L, L)
            r0_v[t, sl] = r0_v[t, sl] + r1_v[t, sl]
        return carry
    lax.fori_loop(0, _T_PER_W, tbody, 0)

    pltpu.sync_copy(r0_v, out_hbm.at[pl.ds(base, _T_PER_W)])


def kernel(x, Wg, bg, W1, b1, W2, b2):
    f32 = jnp.float32
    i32 = jnp.int32

    pos0, pos1, w0, w1, be = pl.pallas_call(
        _gate_body,
        out_shape=[
            jax.ShapeDtypeStruct((N, 1), i32),
            jax.ShapeDtypeStruct((N, 1), i32),
            jax.ShapeDtypeStruct((N, 1), f32),
            jax.ShapeDtypeStruct((N, 1), f32),
            jax.ShapeDtypeStruct((1, NB), i32),
        ],
    )(x, Wg, bg.reshape(1, E))
    pos0 = pos0.reshape(N)
    pos1 = pos1.reshape(N)
    be = be.reshape(NB)

    mesh = plsc.VectorSubcoreMesh(core_axis_name="c", subcore_axis_name="s")
    sc_params = pltpu.CompilerParams(needs_layout_passes=False)

    grid_spec = pltpu.PrefetchScalarGridSpec(
        num_scalar_prefetch=1,
        grid=(NB,),
        in_specs=[
            pl.BlockSpec((1, N), lambda i, be_s: (0, 0)),
            pl.BlockSpec((1, N), lambda i, be_s: (0, 0)),
            pl.BlockSpec((1, N), lambda i, be_s: (0, 0)),
            pl.BlockSpec((1, N), lambda i, be_s: (0, 0)),
            pl.BlockSpec((N, D), lambda i, be_s: (0, 0)),
            pl.BlockSpec((1, D, H), lambda i, be_s: (be_s[i], 0, 0)),
            pl.BlockSpec((1, 1, H), lambda i, be_s: (be_s[i], 0, 0)),
            pl.BlockSpec((1, H, D), lambda i, be_s: (be_s[i], 0, 0)),
            pl.BlockSpec((1, 1, D), lambda i, be_s: (be_s[i], 0, 0)),
        ],
        out_specs=pl.BlockSpec((B, D), lambda i, be_s: (i, 0)),
    )
    ys = pl.pallas_call(
        _moe_body,
        grid_spec=grid_spec,
        out_shape=jax.ShapeDtypeStruct((P, D), f32),
    )(be,
      pos0.reshape(1, N), pos1.reshape(1, N),
      w0.reshape(1, N), w1.reshape(1, N),
      x.astype(jnp.bfloat16),
      W1.astype(jnp.bfloat16), b1.reshape(E, 1, H),
      W2.astype(jnp.bfloat16), b2.reshape(E, 1, D))

    combine_k = pl.kernel(
        _combine_body,
        out_type=jax.ShapeDtypeStruct((N, D), f32),
        mesh=mesh,
        scratch_types=[
            pltpu.VMEM((_T_PER_W,), i32),
            pltpu.VMEM((_T_PER_W,), i32),
            pltpu.VMEM((_T_PER_W, D), f32),
            pltpu.VMEM((_T_PER_W, D), f32),
            pltpu.SemaphoreType.DMA,
            pltpu.SemaphoreType.DMA,
        ],
        compiler_params=sc_params,
    )
    return combine_k(ys, pos0, pos1)


# speedup vs baseline: 1.4181x; 1.2234x over previous
import functools

import jax
import jax.numpy as jnp
from jax import lax
from jax.experimental import pallas as pl
from jax.experimental.pallas import tpu as pltpu
from jax.experimental.pallas import tpu_sc as plsc

N = 2048
D = 768
H = 3072
E = 8
K = 2
B = 128
P = N * K + E * B
NB = P // B
NW = 32
L = 16


def _gelu_tanh(x):
    return 0.5 * x * (1.0 + jnp.tanh(jnp.sqrt(2.0 / jnp.pi) * (x + 0.044715 * x ** 3)))


def _gate_body(x_ref, wg_ref, bg_ref,
               pos0_ref, pos1_ref, w0_ref, w1_ref, be_ref):
    x = x_ref[...]
    logits = lax.dot_general(
        x, wg_ref[...], (((1,), (0,)), ((), ())),
        preferred_element_type=jnp.float32) + bg_ref[...]
    m = jnp.max(logits, axis=-1, keepdims=True)
    ex = jnp.exp(logits - m)
    p = ex / jnp.sum(ex, axis=-1, keepdims=True)

    iota8 = lax.broadcasted_iota(jnp.int32, (N, E), 1)
    m1 = jnp.max(p, axis=-1, keepdims=True)
    a1 = jnp.min(jnp.where(p == m1, iota8, E), axis=-1, keepdims=True)
    oh1 = iota8 == a1
    pm = jnp.where(oh1, -1.0, p)
    m2 = jnp.max(pm, axis=-1, keepdims=True)
    a2 = jnp.min(jnp.where(pm == m2, iota8, E), axis=-1, keepdims=True)
    oh2 = iota8 == a2

    w0_ref[...] = jnp.sum(jnp.where(oh1, p, 0.0), axis=-1, keepdims=True)
    w1_ref[...] = jnp.sum(jnp.where(oh2, p, 0.0), axis=-1, keepdims=True)

    oh = oh1.astype(jnp.int32) + oh2.astype(jnp.int32)
    c = oh
    k = 1
    while k < N:
        c = c + jnp.concatenate(
            [jnp.zeros((k, E), jnp.int32), c[:N - k, :]], axis=0)
        k *= 2
    cexc = c - oh
    counts = c[N - 1:N, :]

    pc = ((counts + (B - 1)) // B) * B
    ends = pc
    k = 1
    while k < E:
        ends = ends + jnp.concatenate(
            [jnp.zeros((1, k), jnp.int32), ends[:, :E - k]], axis=1)
        k *= 2
    opad = ends - pc

    r0 = jnp.sum(jnp.where(oh1, cexc, 0), axis=-1, keepdims=True)
    r1 = jnp.sum(jnp.where(oh2, cexc, 0), axis=-1, keepdims=True)
    off0 = jnp.sum(jnp.where(oh1, opad, 0), axis=-1, keepdims=True)
    off1 = jnp.sum(jnp.where(oh2, opad, 0), axis=-1, keepdims=True)
    pos0_ref[...] = off0 + r0
    pos1_ref[...] = off1 + r1

    bio = lax.broadcasted_iota(jnp.int32, (1, NB), 1) * B
    acc = jnp.zeros((1, NB), jnp.int32)
    for e in range(E):
        acc = acc + (bio >= ends[:, e:e + 1]).astype(jnp.int32)
    be_ref[...] = jnp.minimum(acc, E - 1)


def _moe_body(be_ref, p0_ref, p1_ref, w0_ref, w1_ref, x_ref,
              w1e_ref, b1_ref, w2e_ref, b2_ref, ys_ref):
    i = pl.program_id(0)
    sid = i * B + lax.broadcasted_iota(jnp.int32, (B, 1), 0)
    m0 = p0_ref[...] == sid
    m1 = p1_ref[...] == sid
    gm = (m0 | m1).astype(jnp.float32)
    xb = lax.dot_general(
        gm, x_ref[...], (((1,), (0,)), ((), ())),
        preferred_element_type=jnp.float32)
    sw = jnp.sum(jnp.where(m0, w0_ref[...], 0.0) +
                 jnp.where(m1, w1_ref[...], 0.0), axis=1, keepdims=True)
    h = lax.dot_general(
        xb, w1e_ref[0], (((1,), (0,)), ((), ())),
        preferred_element_type=jnp.float32) + b1_ref[0]
    h = _gelu_tanh(h)
    y = lax.dot_general(
        h, w2e_ref[0], (((1,), (0,)), ((), ())),
        preferred_element_type=jnp.float32) + b2_ref[0]
    ys_ref[...] = y * sw


_T_PER_W = N // NW


def _combine_body(ys_hbm, p0_hbm, p1_hbm, out_hbm,
                  i0_v, i1_v, r0_v, r1_v, s0, s1):
    wid = lax.axis_index("s") * 2 + lax.axis_index("c")
    base = wid * _T_PER_W
    pltpu.sync_copy(p0_hbm.at[pl.ds(base, _T_PER_W)], i0_v)
    pltpu.sync_copy(p1_hbm.at[pl.ds(base, _T_PER_W)], i1_v)
    cp0 = pltpu.async_copy(ys_hbm.at[i0_v], r0_v, s0)
    cp1 = pltpu.async_copy(ys_hbm.at[i1_v], r1_v, s1)
    cp0.wait()
    cp1.wait()

    def tbody(t, carry):
        for j in range(D // L):
            sl = pl.ds(j * L, L)
            r0_v[t, sl] = r0_v[t, sl] + r1_v[t, sl]
        return carry
    lax.fori_loop(0, _T_PER_W, tbody, 0)

    pltpu.sync_copy(r0_v, out_hbm.at[pl.ds(base, _T_PER_W)])


def kernel(x, Wg, bg, W1, b1, W2, b2):
    f32 = jnp.float32
    i32 = jnp.int32

    pos0, pos1, w0, w1, be = pl.pallas_call(
        _gate_body,
        out_shape=[
            jax.ShapeDtypeStruct((N, 1), i32),
            jax.ShapeDtypeStruct((N, 1), i32),
            jax.ShapeDtypeStruct((N, 1), f32),
            jax.ShapeDtypeStruct((N, 1), f32),
            jax.ShapeDtypeStruct((1, NB), i32),
        ],
    )(x, Wg, bg.reshape(1, E))
    pos0 = pos0.reshape(N)
    pos1 = pos1.reshape(N)
    be = be.reshape(NB)

    mesh = plsc.VectorSubcoreMesh(core_axis_name="c", subcore_axis_name="s")
    sc_params = pltpu.CompilerParams(needs_layout_passes=False)

    grid_spec = pltpu.PrefetchScalarGridSpec(
        num_scalar_prefetch=1,
        grid=(NB,),
        in_specs=[
            pl.BlockSpec((1, N), lambda i, be_s: (0, 0)),
            pl.BlockSpec((1, N), lambda i, be_s: (0, 0)),
            pl.BlockSpec((1, N), lambda i, be_s: (0, 0)),
            pl.BlockSpec((1, N), lambda i, be_s: (0, 0)),
            pl.BlockSpec((N, D), lambda i, be_s: (0, 0)),
            pl.BlockSpec((1, D, H), lambda i, be_s: (be_s[i], 0, 0)),
            pl.BlockSpec((1, 1, H), lambda i, be_s: (be_s[i], 0, 0)),
            pl.BlockSpec((1, H, D), lambda i, be_s: (be_s[i], 0, 0)),
            pl.BlockSpec((1, 1, D), lambda i, be_s: (be_s[i], 0, 0)),
        ],
        out_specs=pl.BlockSpec((B, D), lambda i, be_s: (i, 0)),
    )
    ys = pl.pallas_call(
        _moe_body,
        grid_spec=grid_spec,
        out_shape=jax.ShapeDtypeStruct((P, D), f32),
    )(be,
      pos0.reshape(1, N), pos1.reshape(1, N),
      w0.reshape(1, N), w1.reshape(1, N),
      x,
      W1, b1.reshape(E, 1, H),
      W2, b2.reshape(E, 1, D))

    combine_k = pl.kernel(
        _combine_body,
        out_type=jax.ShapeDtypeStruct((N, D), f32),
        mesh=mesh,
        scratch_types=[
            pltpu.VMEM((_T_PER_W,), i32),
            pltpu.VMEM((_T_PER_W,), i32),
            pltpu.VMEM((_T_PER_W, D), f32),
            pltpu.VMEM((_T_PER_W, D), f32),
            pltpu.SemaphoreType.DMA,
            pltpu.SemaphoreType.DMA,
        ],
        compiler_params=sc_params,
    )
    return combine_k(ys, pos0, pos1)


# speedup vs baseline: 1.7575x; 1.2393x over previous
import functools

import jax
import jax.numpy as jnp
from jax import lax
from jax.experimental import pallas as pl
from jax.experimental.pallas import tpu as pltpu
from jax.experimental.pallas import tpu_sc as plsc

N = 2048
D = 768
H = 3072
E = 8
K = 2
B = 256
P = N * K + E * B
NB = P // B
NW = 32
L = 16


def _gelu_tanh(x):
    return 0.5 * x * (1.0 + jnp.tanh(jnp.sqrt(2.0 / jnp.pi) * (x + 0.044715 * x ** 3)))


def _gate_body(x_ref, wg_ref, bg_ref,
               pos0_ref, pos1_ref, w0_ref, w1_ref, be_ref):
    logits = lax.dot_general(
        wg_ref[...], x_ref[...], (((0,), (1,)), ((), ())),
        preferred_element_type=jnp.float32) + bg_ref[...]
    m = jnp.max(logits, axis=0, keepdims=True)
    ex = jnp.exp(logits - m)
    p = ex / jnp.sum(ex, axis=0, keepdims=True)

    iota8 = lax.broadcasted_iota(jnp.int32, (E, N), 0)
    m1 = jnp.max(p, axis=0, keepdims=True)
    a1 = jnp.min(jnp.where(p == m1, iota8, E), axis=0, keepdims=True)
    oh1 = iota8 == a1
    pm = jnp.where(oh1, -1.0, p)
    m2 = jnp.max(pm, axis=0, keepdims=True)
    a2 = jnp.min(jnp.where(pm == m2, iota8, E), axis=0, keepdims=True)
    oh2 = iota8 == a2

    w0_ref[...] = jnp.sum(jnp.where(oh1, p, 0.0), axis=0, keepdims=True)
    w1_ref[...] = jnp.sum(jnp.where(oh2, p, 0.0), axis=0, keepdims=True)

    oh = oh1.astype(jnp.int32) + oh2.astype(jnp.int32)
    c = oh
    k = 1
    while k < N:
        c = c + jnp.concatenate(
            [jnp.zeros((E, k), jnp.int32), c[:, :N - k]], axis=1)
        k *= 2
    cexc = c - oh
    counts = c[:, N - 1:N]

    pc = ((counts + (B - 1)) // B) * B
    ends = pc
    k = 1
    while k < E:
        ends = ends + jnp.concatenate(
            [jnp.zeros((k, 1), jnp.int32), ends[:E - k, :]], axis=0)
        k *= 2
    opad = ends - pc

    r0 = jnp.sum(jnp.where(oh1, cexc, 0), axis=0, keepdims=True)
    r1 = jnp.sum(jnp.where(oh2, cexc, 0), axis=0, keepdims=True)
    off0 = jnp.sum(jnp.where(oh1, opad, 0), axis=0, keepdims=True)
    off1 = jnp.sum(jnp.where(oh2, opad, 0), axis=0, keepdims=True)
    pos0_ref[...] = off0 + r0
    pos1_ref[...] = off1 + r1

    bio = lax.broadcasted_iota(jnp.int32, (1, NB + 1), 1) * B
    acc = jnp.sum((bio >= ends).astype(jnp.int32), axis=0, keepdims=True)
    used = jnp.sum(pc, axis=0, keepdims=True) // B
    is_last = lax.broadcasted_iota(jnp.int32, (1, NB + 1), 1) == NB
    be_ref[...] = jnp.where(is_last, used, jnp.minimum(acc, E - 1))


def _moe_body(be_ref, p0_ref, p1_ref, w0_ref, w1_ref, x_ref,
              w1e_ref, b1_ref, w2e_ref, b2_ref, ys_ref):
    i = pl.program_id(0)

    @pl.when(i < be_ref[NB])
    def _():
        sid = i * B + lax.broadcasted_iota(jnp.int32, (B, 1), 0)
        m0 = p0_ref[...] == sid
        m1 = p1_ref[...] == sid
        gm = (m0 | m1).astype(jnp.float32)
        xb = lax.dot_general(
            gm, x_ref[...], (((1,), (0,)), ((), ())),
            preferred_element_type=jnp.float32)
        sw = jnp.sum(jnp.where(m0, w0_ref[...], 0.0) +
                     jnp.where(m1, w1_ref[...], 0.0), axis=1, keepdims=True)
        h = lax.dot_general(
            xb, w1e_ref[0], (((1,), (0,)), ((), ())),
            preferred_element_type=jnp.float32) + b1_ref[0]
        h = _gelu_tanh(h)
        y = lax.dot_general(
            h, w2e_ref[0], (((1,), (0,)), ((), ())),
            preferred_element_type=jnp.float32) + b2_ref[0]
        ys_ref[...] = y * sw


_T_PER_W = N // NW


def _combine_body(ys_hbm, p0_hbm, p1_hbm, out_hbm,
                  i0_v, i1_v, r0_v, r1_v, s0, s1):
    wid = lax.axis_index("s") * 2 + lax.axis_index("c")
    base = wid * _T_PER_W
    pltpu.sync_copy(p0_hbm.at[0, pl.ds(base, _T_PER_W)], i0_v)
    pltpu.sync_copy(p1_hbm.at[0, pl.ds(base, _T_PER_W)], i1_v)
    cp0 = pltpu.async_copy(ys_hbm.at[i0_v], r0_v, s0)
    cp1 = pltpu.async_copy(ys_hbm.at[i1_v], r1_v, s1)
    cp0.wait()
    cp1.wait()

    def tbody(t, carry):
        for j in range(D // L):
            sl = pl.ds(j * L, L)
            r0_v[t, sl] = r0_v[t, sl] + r1_v[t, sl]
        return carry
    lax.fori_loop(0, _T_PER_W, tbody, 0)

    pltpu.sync_copy(r0_v, out_hbm.at[pl.ds(base, _T_PER_W)])


def kernel(x, Wg, bg, W1, b1, W2, b2):
    f32 = jnp.float32
    i32 = jnp.int32

    pos0, pos1, w0, w1, be = pl.pallas_call(
        _gate_body,
        out_shape=[
            jax.ShapeDtypeStruct((1, N), i32),
            jax.ShapeDtypeStruct((1, N), i32),
            jax.ShapeDtypeStruct((1, N), f32),
            jax.ShapeDtypeStruct((1, N), f32),
            jax.ShapeDtypeStruct((1, NB + 1), i32),
        ],
    )(x, Wg, bg.reshape(E, 1))

    grid_spec = pltpu.PrefetchScalarGridSpec(
        num_scalar_prefetch=1,
        grid=(NB,),
        in_specs=[
            pl.BlockSpec((1, N), lambda i, be_s: (0, 0)),
            pl.BlockSpec((1, N), lambda i, be_s: (0, 0)),
            pl.BlockSpec((1, N), lambda i, be_s: (0, 0)),
            pl.BlockSpec((1, N), lambda i, be_s: (0, 0)),
            pl.BlockSpec((N, D), lambda i, be_s: (0, 0)),
            pl.BlockSpec((1, D, H), lambda i, be_s: (be_s[i], 0, 0)),
            pl.BlockSpec((1, 1, H), lambda i, be_s: (be_s[i], 0, 0)),
            pl.BlockSpec((1, H, D), lambda i, be_s: (be_s[i], 0, 0)),
            pl.BlockSpec((1, 1, D), lambda i, be_s: (be_s[i], 0, 0)),
        ],
        out_specs=pl.BlockSpec((B, D), lambda i, be_s: (i, 0)),
    )
    ys = pl.pallas_call(
        _moe_body,
        grid_spec=grid_spec,
        out_shape=jax.ShapeDtypeStruct((P, D), f32),
    )(be.reshape(NB + 1), pos0, pos1, w0, w1, x,
      W1, b1.reshape(E, 1, H), W2, b2.reshape(E, 1, D))

    mesh = plsc.VectorSubcoreMesh(core_axis_name="c", subcore_axis_name="s")
    combine_k = pl.kernel(
        _combine_body,
        out_type=jax.ShapeDtypeStruct((N, D), f32),
        mesh=mesh,
        scratch_types=[
            pltpu.VMEM((_T_PER_W,), i32),
            pltpu.VMEM((_T_PER_W,), i32),
            pltpu.VMEM((_T_PER_W, D), f32),
            pltpu.VMEM((_T_PER_W, D), f32),
            pltpu.SemaphoreType.DMA,
            pltpu.SemaphoreType.DMA,
        ],
        compiler_params=pltpu.CompilerParams(needs_layout_passes=False),
    )
    return combine_k(ys, pos0, pos1)
